# baseline (speedup 1.0000x reference)
import functools

import jax
import jax.numpy as jnp
from jax import lax
from jax.experimental import pallas as pl
from jax.experimental.pallas import tpu as pltpu
from jax.experimental.pallas import tpu_sc as plsc

N = 10000
D = 128
G = 64
NPAD = 10240
CHUNK = 112
NBLK = 9
NCORES = 2
NTILES = NCORES * 16


def _sc_agg(x, idx4d, zeros_blk):
    nblocks = idx4d.shape[1]
    rows_per_tile = NPAD // 16
    mesh = plsc.VectorSubcoreMesh(core_axis_name="c", subcore_axis_name="s",
                                  num_cores=NCORES)

    @functools.partial(
        pl.kernel,
        mesh=mesh,
        out_type=jax.ShapeDtypeStruct((NCORES, NPAD, D), jnp.float32),
        scratch_types=[
            pltpu.VMEM((2 * NBLK, CHUNK), jnp.int32),
            pltpu.VMEM((CHUNK, D), jnp.float32),
            pltpu.VMEM((CHUNK, D), jnp.float32),
            pltpu.VMEM((CHUNK, D), jnp.float32),
            pltpu.VMEM_SHARED((NPAD, D), jnp.float32),
            pltpu.SemaphoreType.DMA,
            pltpu.SemaphoreType.DMA,
            pltpu.SemaphoreType.DMA,
            pltpu.SemaphoreType.DMA,
            pltpu.SemaphoreType.DMA,
            pltpu.SemaphoreType.DMA,
        ],
    )
    def agg_kernel(x_hbm, idx_hbm, z_hbm, out_hbm,
                   idx_v, rows_a, rows_b, rows_c, acc_sh,
                   gsem_a, gsem_b, gsem_c, ssem_a, ssem_b, ssem_c):
        c = lax.axis_index("c")
        s = lax.axis_index("s")
        wid = c * 16 + s
        rows = (rows_a, rows_b, rows_c)
        gsems = (gsem_a, gsem_b, gsem_c)
        ssems = (ssem_a, ssem_b, ssem_c)
        slices = [(k * CHUNK, CHUNK) for k in range(rows_per_tile // CHUNK)]
        rem = rows_per_tile - len(slices) * CHUNK
        if rem:
            slices.append((len(slices) * CHUNK, rem))
        r0 = s * rows_per_tile
        pltpu.sync_copy(z_hbm, rows_a)
        for off, sz in slices:
            pltpu.sync_copy(rows_a.at[pl.ds(0, sz)],
                            acc_sh.at[pl.ds(r0 + off, sz)])
        plsc.subcore_barrier()

        def outer(b, carry):
            pltpu.sync_copy(idx_hbm.at[wid, b], idx_v)
            g = [None] * NBLK
            sc = [None] * NBLK
            for j in range(2):
                g[j] = pltpu.async_copy(
                    x_hbm.at[idx_v.at[j]], rows[j % 3], gsems[j % 3])
            for j in range(NBLK):
                g[j].wait()
                sc[j] = pltpu.async_copy(
                    rows[j % 3], acc_sh.at[idx_v.at[NBLK + j]],
                    ssems[j % 3], add=True)
                if j + 2 < NBLK:
                    if j >= 1:
                        sc[j - 1].wait()
                    g[j + 2] = pltpu.async_copy(
                        x_hbm.at[idx_v.at[j + 2]], rows[(j + 2) % 3],
                        gsems[(j + 2) % 3])
            for j in range(max(NBLK - 3, 0), NBLK):
                sc[j].wait()
            return carry

        lax.fori_loop(0, nblocks, outer, 0)
        plsc.subcore_barrier()
        prev = None
        for k, (off, sz) in enumerate(slices):
            pltpu.async_copy(acc_sh.at[pl.ds(r0 + off, sz)],
                             rows[k % 2].at[pl.ds(0, sz)], gsems[k % 2]).wait()
            if prev is not None:
                prev.wait()
            prev = pltpu.async_copy(rows[k % 2].at[pl.ds(0, sz)],
                                    out_hbm.at[c, pl.ds(r0 + off, sz)],
                                    ssems[k % 2])
        prev.wait()

    return agg_kernel(x, idx4d, zeros_blk)


def _tc_layer(x, parts, W1, b1, W2, b2, gamma, beta):
    def body(x_ref, p_ref, w1_ref, b1_ref, w2_ref, b2_ref, g_ref, bt_ref, o_ref):
        h = x_ref[...] + p_ref[0, :N, :]
        for c in range(1, NCORES):
            h = h + p_ref[c, :N, :]
        t = jnp.maximum(
            jnp.dot(h, w1_ref[...], preferred_element_type=jnp.float32)
            + b1_ref[...], 0.0)
        h2 = (jnp.dot(t, w2_ref[...], preferred_element_type=jnp.float32)
              + b2_ref[...])
        mean = jnp.mean(h2, axis=0, keepdims=True)
        cen = h2 - mean
        var = jnp.mean(cen * cen, axis=0, keepdims=True)
        h3 = cen * lax.rsqrt(var + 1e-5) * g_ref[...] + bt_ref[...]
        o_ref[...] = jnp.maximum(h3, 0.0)

    return pl.pallas_call(
        body,
        out_shape=jax.ShapeDtypeStruct((N, D), jnp.float32),
    )(x, parts, W1, b1.reshape(1, D), W2, b2.reshape(1, D),
      gamma.reshape(1, D), beta.reshape(1, D))


def _tc_final(x, parts, W1, b1, W2, b2, gamma, beta, batch2d,
              Wr1, br1, Wr2, br2, Wr3p, br3p):
    def body(x_ref, p_ref, w1_ref, b1_ref, w2_ref, b2_ref, g_ref, bt_ref,
             batch_ref, wr1_ref, br1_ref, wr2_ref, br2_ref, wr3_ref, br3_ref,
             o_ref):
        h = x_ref[...] + p_ref[0, :N, :]
        for c in range(1, NCORES):
            h = h + p_ref[c, :N, :]
        t = jnp.maximum(
            jnp.dot(h, w1_ref[...], preferred_element_type=jnp.float32)
            + b1_ref[...], 0.0)
        h2 = (jnp.dot(t, w2_ref[...], preferred_element_type=jnp.float32)
              + b2_ref[...])
        mean = jnp.mean(h2, axis=0, keepdims=True)
        cen = h2 - mean
        var = jnp.mean(cen * cen, axis=0, keepdims=True)
        h3 = cen * lax.rsqrt(var + 1e-5) * g_ref[...] + bt_ref[...]
        x3 = jnp.maximum(h3, 0.0)
        seg = lax.broadcasted_iota(jnp.int32, (G, N), 0)
        onehot = (seg == batch_ref[...]).astype(jnp.float32)
        pooled = jnp.dot(onehot, x3, preferred_element_type=jnp.float32)
        r = jnp.maximum(
            jnp.dot(pooled, wr1_ref[...], preferred_element_type=jnp.float32)
            + br1_ref[...], 0.0)
        r = jnp.maximum(
            jnp.dot(r, wr2_ref[...], preferred_element_type=jnp.float32)
            + br2_ref[...], 0.0)
        o_ref[...] = (jnp.dot(r, wr3_ref[...], preferred_element_type=jnp.float32)
                      + br3_ref[...])

    return pl.pallas_call(
        body,
        out_shape=jax.ShapeDtypeStruct((G, 128), jnp.float32),
    )(x, parts, W1, b1.reshape(1, D), W2, b2.reshape(1, D),
      gamma.reshape(1, D), beta.reshape(1, D), batch2d,
      Wr1, br1.reshape(1, 512), Wr2, br2.reshape(1, 256), Wr3p, br3p)


def kernel(x, edge_index, batch,
           W1_0, b1_0, W2_0, b2_0, gamma_0, beta_0,
           W1_1, b1_1, W2_1, b2_1, gamma_1, beta_1,
           W1_2, b1_2, W2_2, b2_2, gamma_2, beta_2,
           Wr1, br1, Wr2, br2, Wr3, br3):
    E = edge_index.shape[1]
    nblocks = 20
    e_pad = NTILES * nblocks * NBLK * CHUNK
    src_p = jnp.concatenate(
        [edge_index[0], jnp.zeros((e_pad - E,), jnp.int32)])
    dst_p = jnp.concatenate(
        [edge_index[1], jnp.full((e_pad - E,), N + 200, jnp.int32)])
    src4d = src_p.reshape(NTILES, nblocks, NBLK, CHUNK)
    dst4d = dst_p.reshape(NTILES, nblocks, NBLK, CHUNK)
    idx4d = jnp.concatenate([src4d, dst4d], axis=2)
    zeros_blk = jnp.zeros((CHUNK, D), jnp.float32)
    batch2d = batch.reshape(1, N)
    T = Wr3.shape[1]
    Wr3p = jnp.pad(Wr3, ((0, 0), (0, 128 - T)))
    br3p = jnp.pad(br3, (0, 128 - T)).reshape(1, 128)

    p0 = _sc_agg(x, idx4d, zeros_blk)
    x1 = _tc_layer(x, p0, W1_0, b1_0, W2_0, b2_0, gamma_0, beta_0)
    p1 = _sc_agg(x1, idx4d, zeros_blk)
    x2 = _tc_layer(x1, p1, W1_1, b1_1, W2_1, b2_1, gamma_1, beta_1)
    p2 = _sc_agg(x2, idx4d, zeros_blk)
    out = _tc_final(x2, p2, W1_2, b1_2, W2_2, b2_2, gamma_2, beta_2,
                    batch2d, Wr1, br1, Wr2, br2, Wr3p, br3p)
    return out[:, :T]


# speedup vs baseline: 1.0032x; 1.0032x over previous
import functools

import jax
import jax.numpy as jnp
from jax import lax
from jax.experimental import pallas as pl
from jax.experimental.pallas import tpu as pltpu
from jax.experimental.pallas import tpu_sc as plsc

N = 10000
D = 128
G = 64
NPAD = 10240
CHUNK = 112
NBLK = 9
NCORES = 2
NTILES = NCORES * 16


def _sc_agg(x, idx4d, zeros_blk):
    nblocks = idx4d.shape[1]
    rows_per_tile = NPAD // 16
    mesh = plsc.VectorSubcoreMesh(core_axis_name="c", subcore_axis_name="s",
                                  num_cores=NCORES)

    @functools.partial(
        pl.kernel,
        mesh=mesh,
        out_type=jax.ShapeDtypeStruct((NCORES, NPAD, D), jnp.float32),
        scratch_types=[
            pltpu.VMEM((2 * NBLK, CHUNK), jnp.int32),
            pltpu.VMEM((CHUNK, D), jnp.float32),
            pltpu.VMEM((CHUNK, D), jnp.float32),
            pltpu.VMEM((CHUNK, D), jnp.float32),
            pltpu.VMEM_SHARED((NPAD, D), jnp.float32),
            pltpu.SemaphoreType.DMA,
            pltpu.SemaphoreType.DMA,
            pltpu.SemaphoreType.DMA,
            pltpu.SemaphoreType.DMA,
            pltpu.SemaphoreType.DMA,
            pltpu.SemaphoreType.DMA,
        ],
    )
    def agg_kernel(x_hbm, idx_hbm, z_hbm, out_hbm,
                   idx_v, rows_a, rows_b, rows_c, acc_sh,
                   gsem_a, gsem_b, gsem_c, ssem_a, ssem_b, ssem_c):
        c = lax.axis_index("c")
        s = lax.axis_index("s")
        wid = c * 16 + s
        rows = (rows_a, rows_b, rows_c)
        gsems = (gsem_a, gsem_b, gsem_c)
        ssems = (ssem_a, ssem_b, ssem_c)
        slices = [(k * CHUNK, CHUNK) for k in range(rows_per_tile // CHUNK)]
        rem = rows_per_tile - len(slices) * CHUNK
        if rem:
            slices.append((len(slices) * CHUNK, rem))
        r0 = s * rows_per_tile
        pltpu.sync_copy(z_hbm, rows_a)
        for off, sz in slices:
            pltpu.sync_copy(rows_a.at[pl.ds(0, sz)],
                            acc_sh.at[pl.ds(r0 + off, sz)])
        plsc.subcore_barrier()

        def outer(b, carry):
            pltpu.sync_copy(idx_hbm.at[wid, b], idx_v)
            g = [None] * NBLK
            for j in range(2):
                g[j] = pltpu.async_copy(
                    x_hbm.at[idx_v.at[j]], rows[j % 3], gsems[j % 3])
            for j in range(NBLK):
                if j + 2 < NBLK:
                    g[j + 2] = pltpu.async_copy(
                        x_hbm.at[idx_v.at[j + 2]], rows[(j + 2) % 3],
                        gsems[(j + 2) % 3])
                g[j].wait()
                pltpu.sync_copy(rows[j % 3], acc_sh.at[idx_v.at[NBLK + j]],
                                add=True)
            return carry

        lax.fori_loop(0, nblocks, outer, 0)
        plsc.subcore_barrier()
        prev = None
        for k, (off, sz) in enumerate(slices):
            pltpu.async_copy(acc_sh.at[pl.ds(r0 + off, sz)],
                             rows[k % 2].at[pl.ds(0, sz)], gsems[k % 2]).wait()
            if prev is not None:
                prev.wait()
            prev = pltpu.async_copy(rows[k % 2].at[pl.ds(0, sz)],
                                    out_hbm.at[c, pl.ds(r0 + off, sz)],
                                    ssems[k % 2])
        prev.wait()

    return agg_kernel(x, idx4d, zeros_blk)


def _tc_layer(x, parts, W1, b1, W2, b2, gamma, beta):
    def body(x_ref, p_ref, w1_ref, b1_ref, w2_ref, b2_ref, g_ref, bt_ref, o_ref):
        h = x_ref[...] + p_ref[0, :N, :]
        for c in range(1, NCORES):
            h = h + p_ref[c, :N, :]
        t = jnp.maximum(
            jnp.dot(h, w1_ref[...], preferred_element_type=jnp.float32)
            + b1_ref[...], 0.0)
        h2 = (jnp.dot(t, w2_ref[...], preferred_element_type=jnp.float32)
              + b2_ref[...])
        mean = jnp.mean(h2, axis=0, keepdims=True)
        cen = h2 - mean
        var = jnp.mean(cen * cen, axis=0, keepdims=True)
        h3 = cen * lax.rsqrt(var + 1e-5) * g_ref[...] + bt_ref[...]
        o_ref[...] = jnp.maximum(h3, 0.0)

    return pl.pallas_call(
        body,
        out_shape=jax.ShapeDtypeStruct((N, D), jnp.float32),
    )(x, parts, W1, b1.reshape(1, D), W2, b2.reshape(1, D),
      gamma.reshape(1, D), beta.reshape(1, D))


def _tc_final(x, parts, W1, b1, W2, b2, gamma, beta, batch2d,
              Wr1, br1, Wr2, br2, Wr3p, br3p):
    def body(x_ref, p_ref, w1_ref, b1_ref, w2_ref, b2_ref, g_ref, bt_ref,
             batch_ref, wr1_ref, br1_ref, wr2_ref, br2_ref, wr3_ref, br3_ref,
             o_ref):
        h = x_ref[...] + p_ref[0, :N, :]
        for c in range(1, NCORES):
            h = h + p_ref[c, :N, :]
        t = jnp.maximum(
            jnp.dot(h, w1_ref[...], preferred_element_type=jnp.float32)
            + b1_ref[...], 0.0)
        h2 = (jnp.dot(t, w2_ref[...], preferred_element_type=jnp.float32)
              + b2_ref[...])
        mean = jnp.mean(h2, axis=0, keepdims=True)
        cen = h2 - mean
        var = jnp.mean(cen * cen, axis=0, keepdims=True)
        h3 = cen * lax.rsqrt(var + 1e-5) * g_ref[...] + bt_ref[...]
        x3 = jnp.maximum(h3, 0.0)
        seg = lax.broadcasted_iota(jnp.int32, (G, N), 0)
        onehot = (seg == batch_ref[...]).astype(jnp.float32)
        pooled = jnp.dot(onehot, x3, preferred_element_type=jnp.float32)
        r = jnp.maximum(
            jnp.dot(pooled, wr1_ref[...], preferred_element_type=jnp.float32)
            + br1_ref[...], 0.0)
        r = jnp.maximum(
            jnp.dot(r, wr2_ref[...], preferred_element_type=jnp.float32)
            + br2_ref[...], 0.0)
        o_ref[...] = (jnp.dot(r, wr3_ref[...], preferred_element_type=jnp.float32)
                      + br3_ref[...])

    return pl.pallas_call(
        body,
        out_shape=jax.ShapeDtypeStruct((G, 128), jnp.float32),
    )(x, parts, W1, b1.reshape(1, D), W2, b2.reshape(1, D),
      gamma.reshape(1, D), beta.reshape(1, D), batch2d,
      Wr1, br1.reshape(1, 512), Wr2, br2.reshape(1, 256), Wr3p, br3p)


def kernel(x, edge_index, batch,
           W1_0, b1_0, W2_0, b2_0, gamma_0, beta_0,
           W1_1, b1_1, W2_1, b2_1, gamma_1, beta_1,
           W1_2, b1_2, W2_2, b2_2, gamma_2, beta_2,
           Wr1, br1, Wr2, br2, Wr3, br3):
    E = edge_index.shape[1]
    nblocks = 20
    e_pad = NTILES * nblocks * NBLK * CHUNK
    src_p = jnp.concatenate(
        [edge_index[0], jnp.zeros((e_pad - E,), jnp.int32)])
    dst_p = jnp.concatenate(
        [edge_index[1], jnp.full((e_pad - E,), N + 200, jnp.int32)])
    src4d = src_p.reshape(NTILES, nblocks, NBLK, CHUNK)
    dst4d = dst_p.reshape(NTILES, nblocks, NBLK, CHUNK)
    idx4d = jnp.concatenate([src4d, dst4d], axis=2)
    zeros_blk = jnp.zeros((CHUNK, D), jnp.float32)
    batch2d = batch.reshape(1, N)
    T = Wr3.shape[1]
    Wr3p = jnp.pad(Wr3, ((0, 0), (0, 128 - T)))
    br3p = jnp.pad(br3, (0, 128 - T)).reshape(1, 128)

    p0 = _sc_agg(x, idx4d, zeros_blk)
    x1 = _tc_layer(x, p0, W1_0, b1_0, W2_0, b2_0, gamma_0, beta_0)
    p1 = _sc_agg(x1, idx4d, zeros_blk)
    x2 = _tc_layer(x1, p1, W1_1, b1_1, W2_1, b2_1, gamma_1, beta_1)
    p2 = _sc_agg(x2, idx4d, zeros_blk)
    out = _tc_final(x2, p2, W1_2, b1_2, W2_2, b2_2, gamma_2, beta_2,
                    batch2d, Wr1, br1, Wr2, br2, Wr3p, br3p)
    return out[:, :T]


# speedup vs baseline: 1.9187x; 1.9126x over previous
import functools

import jax
import jax.numpy as jnp
from jax import lax
from jax.experimental import pallas as pl
from jax.experimental.pallas import tpu as pltpu
from jax.experimental.pallas import tpu_sc as plsc

N = 10000
D = 128
G = 64
NPAD = 10240
CHUNK = 80
NBLK = 10
NCORES = 2
NTILES = NCORES * 16


def _sc_agg(x, idx4d, zeros_blk):
    nblocks = idx4d.shape[1]
    rows_per_tile = NPAD // 16
    mesh = plsc.VectorSubcoreMesh(core_axis_name="c", subcore_axis_name="s",
                                  num_cores=NCORES)

    @functools.partial(
        pl.kernel,
        mesh=mesh,
        out_type=jax.ShapeDtypeStruct((NCORES, NPAD, D), jnp.float32),
        scratch_types=[
            pltpu.VMEM((2 * NBLK, CHUNK), jnp.int32),
            pltpu.VMEM((CHUNK, D), jnp.float32),
            pltpu.VMEM((CHUNK, D), jnp.float32),
            pltpu.VMEM((CHUNK, D), jnp.float32),
            pltpu.VMEM_SHARED((NPAD, D), jnp.float32),
            pltpu.SemaphoreType.DMA,
            pltpu.SemaphoreType.DMA,
            pltpu.SemaphoreType.DMA,
            pltpu.SemaphoreType.DMA,
            pltpu.SemaphoreType.DMA,
            pltpu.SemaphoreType.DMA,
        ],
    )
    def agg_kernel(x_hbm, idx_hbm, z_hbm, out_hbm,
                   idx_v, rows_a, rows_b, rows_c, acc_sh,
                   gsem_a, gsem_b, gsem_c, ssem_a, ssem_b, ssem_c):
        c = lax.axis_index("c")
        s = lax.axis_index("s")
        wid = c * 16 + s
        rows = (rows_a, rows_b, rows_c)
        gsems = (gsem_a, gsem_b, gsem_c)
        ssems = (ssem_a, ssem_b, ssem_c)
        slices = [(k * CHUNK, CHUNK) for k in range(rows_per_tile // CHUNK)]
        rem = rows_per_tile - len(slices) * CHUNK
        if rem:
            slices.append((len(slices) * CHUNK, rem))
        r0 = s * rows_per_tile
        pltpu.sync_copy(z_hbm, rows_a)
        for off, sz in slices:
            pltpu.sync_copy(rows_a.at[pl.ds(0, sz)],
                            acc_sh.at[pl.ds(r0 + off, sz)])
        plsc.subcore_barrier()

        def outer(b, carry):
            pltpu.sync_copy(idx_hbm.at[wid, b], idx_v)
            g = [None] * NBLK
            for j in range(2):
                g[j] = pltpu.async_copy(
                    x_hbm.at[idx_v.at[j]], rows[j % 3], gsems[j % 3])
            for j in range(NBLK):
                if j + 2 < NBLK:
                    g[j + 2] = pltpu.async_copy(
                        x_hbm.at[idx_v.at[j + 2]], rows[(j + 2) % 3],
                        gsems[(j + 2) % 3])
                g[j].wait()
                pltpu.sync_copy(rows[j % 3], acc_sh.at[idx_v.at[NBLK + j]],
                                add=True)
            return carry

        lax.fori_loop(0, nblocks, outer, 0)
        plsc.subcore_barrier()
        prev = None
        for k, (off, sz) in enumerate(slices):
            pltpu.async_copy(acc_sh.at[pl.ds(r0 + off, sz)],
                             rows[k % 2].at[pl.ds(0, sz)], gsems[k % 2]).wait()
            if prev is not None:
                prev.wait()
            prev = pltpu.async_copy(rows[k % 2].at[pl.ds(0, sz)],
                                    out_hbm.at[c, pl.ds(r0 + off, sz)],
                                    ssems[k % 2])
        prev.wait()

    return agg_kernel(x, idx4d, zeros_blk)


def _tc_layer(x, parts, W1, b1, W2, b2, gamma, beta):
    def body(x_ref, p_ref, w1_ref, b1_ref, w2_ref, b2_ref, g_ref, bt_ref, o_ref):
        h = x_ref[...] + p_ref[0, :N, :]
        for c in range(1, NCORES):
            h = h + p_ref[c, :N, :]
        t = jnp.maximum(
            jnp.dot(h, w1_ref[...], preferred_element_type=jnp.float32)
            + b1_ref[...], 0.0)
        h2 = (jnp.dot(t, w2_ref[...], preferred_element_type=jnp.float32)
              + b2_ref[...])
        mean = jnp.mean(h2, axis=0, keepdims=True)
        cen = h2 - mean
        var = jnp.mean(cen * cen, axis=0, keepdims=True)
        h3 = cen * lax.rsqrt(var + 1e-5) * g_ref[...] + bt_ref[...]
        o_ref[...] = jnp.maximum(h3, 0.0)

    return pl.pallas_call(
        body,
        out_shape=jax.ShapeDtypeStruct((N, D), jnp.float32),
    )(x, parts, W1, b1.reshape(1, D), W2, b2.reshape(1, D),
      gamma.reshape(1, D), beta.reshape(1, D))


def _tc_final(x, parts, W1, b1, W2, b2, gamma, beta, batch2d,
              Wr1, br1, Wr2, br2, Wr3p, br3p):
    def body(x_ref, p_ref, w1_ref, b1_ref, w2_ref, b2_ref, g_ref, bt_ref,
             batch_ref, wr1_ref, br1_ref, wr2_ref, br2_ref, wr3_ref, br3_ref,
             o_ref):
        h = x_ref[...] + p_ref[0, :N, :]
        for c in range(1, NCORES):
            h = h + p_ref[c, :N, :]
        t = jnp.maximum(
            jnp.dot(h, w1_ref[...], preferred_element_type=jnp.float32)
            + b1_ref[...], 0.0)
        h2 = (jnp.dot(t, w2_ref[...], preferred_element_type=jnp.float32)
              + b2_ref[...])
        mean = jnp.mean(h2, axis=0, keepdims=True)
        cen = h2 - mean
        var = jnp.mean(cen * cen, axis=0, keepdims=True)
        h3 = cen * lax.rsqrt(var + 1e-5) * g_ref[...] + bt_ref[...]
        x3 = jnp.maximum(h3, 0.0)
        seg = lax.broadcasted_iota(jnp.int32, (G, N), 0)
        onehot = (seg == batch_ref[...]).astype(jnp.float32)
        pooled = jnp.dot(onehot, x3, preferred_element_type=jnp.float32)
        r = jnp.maximum(
            jnp.dot(pooled, wr1_ref[...], preferred_element_type=jnp.float32)
            + br1_ref[...], 0.0)
        r = jnp.maximum(
            jnp.dot(r, wr2_ref[...], preferred_element_type=jnp.float32)
            + br2_ref[...], 0.0)
        o_ref[...] = (jnp.dot(r, wr3_ref[...], preferred_element_type=jnp.float32)
                      + br3_ref[...])

    return pl.pallas_call(
        body,
        out_shape=jax.ShapeDtypeStruct((G, 128), jnp.float32),
    )(x, parts, W1, b1.reshape(1, D), W2, b2.reshape(1, D),
      gamma.reshape(1, D), beta.reshape(1, D), batch2d,
      Wr1, br1.reshape(1, 512), Wr2, br2.reshape(1, 256), Wr3p, br3p)


def kernel(x, edge_index, batch,
           W1_0, b1_0, W2_0, b2_0, gamma_0, beta_0,
           W1_1, b1_1, W2_1, b2_1, gamma_1, beta_1,
           W1_2, b1_2, W2_2, b2_2, gamma_2, beta_2,
           Wr1, br1, Wr2, br2, Wr3, br3):
    E = edge_index.shape[1]
    nblocks = 25
    e_pad = NTILES * nblocks * NBLK * CHUNK
    src_p = jnp.concatenate(
        [edge_index[0], jnp.zeros((e_pad - E,), jnp.int32)])
    dst_p = jnp.concatenate(
        [edge_index[1], jnp.full((e_pad - E,), N + 200, jnp.int32)])
    src4d = src_p.reshape(NTILES, nblocks, NBLK, CHUNK)
    dst4d = dst_p.reshape(NTILES, nblocks, NBLK, CHUNK)
    idx4d = jnp.concatenate([src4d, dst4d], axis=2)
    zeros_blk = jnp.zeros((CHUNK, D), jnp.float32)
    batch2d = batch.reshape(1, N)
    T = Wr3.shape[1]
    Wr3p = jnp.pad(Wr3, ((0, 0), (0, 128 - T)))
    br3p = jnp.pad(br3, (0, 128 - T)).reshape(1, 128)

    p0 = _sc_agg(x, idx4d, zeros_blk)
    x1 = _tc_layer(x, p0, W1_0, b1_0, W2_0, b2_0, gamma_0, beta_0)
    p1 = _sc_agg(x1, idx4d, zeros_blk)
    x2 = _tc_layer(x1, p1, W1_1, b1_1, W2_1, b2_1, gamma_1, beta_1)
    p2 = _sc_agg(x2, idx4d, zeros_blk)
    out = _tc_final(x2, p2, W1_2, b1_2, W2_2, b2_2, gamma_2, beta_2,
                    batch2d, Wr1, br1, Wr2, br2, Wr3p, br3p)
    return out[:, :T]


# speedup vs baseline: 2.2082x; 1.1509x over previous
import functools

import jax
import jax.numpy as jnp
from jax import lax
from jax.experimental import pallas as pl
from jax.experimental.pallas import tpu as pltpu
from jax.experimental.pallas import tpu_sc as plsc

N = 10000
D = 128
G = 64
NPAD = 10240
CHUNK = 80
NBLK = 25
NCORES = 2
NTILES = NCORES * 16


def _sc_agg(x, idx4d, zeros_blk):
    nblocks = idx4d.shape[1]
    rows_per_tile = NPAD // 16
    mesh = plsc.VectorSubcoreMesh(core_axis_name="c", subcore_axis_name="s",
                                  num_cores=NCORES)

    @functools.partial(
        pl.kernel,
        mesh=mesh,
        out_type=jax.ShapeDtypeStruct((NCORES, NPAD, D), jnp.float32),
        scratch_types=[
            pltpu.VMEM((2 * NBLK, CHUNK), jnp.int32),
            pltpu.VMEM((CHUNK, D), jnp.float32),
            pltpu.VMEM((CHUNK, D), jnp.float32),
            pltpu.VMEM((CHUNK, D), jnp.float32),
            pltpu.VMEM_SHARED((NPAD, D), jnp.float32),
            pltpu.SemaphoreType.DMA,
            pltpu.SemaphoreType.DMA,
            pltpu.SemaphoreType.DMA,
            pltpu.SemaphoreType.DMA,
            pltpu.SemaphoreType.DMA,
            pltpu.SemaphoreType.DMA,
        ],
    )
    def agg_kernel(x_hbm, idx_hbm, z_hbm, out_hbm,
                   idx_v, rows_a, rows_b, rows_c, acc_sh,
                   gsem_a, gsem_b, gsem_c, ssem_a, ssem_b, ssem_c):
        c = lax.axis_index("c")
        s = lax.axis_index("s")
        wid = c * 16 + s
        rows = (rows_a, rows_b, rows_c)
        gsems = (gsem_a, gsem_b, gsem_c)
        ssems = (ssem_a, ssem_b, ssem_c)
        slices = [(k * CHUNK, CHUNK) for k in range(rows_per_tile // CHUNK)]
        rem = rows_per_tile - len(slices) * CHUNK
        if rem:
            slices.append((len(slices) * CHUNK, rem))
        r0 = s * rows_per_tile
        pltpu.sync_copy(z_hbm, rows_a)
        for off, sz in slices:
            pltpu.sync_copy(rows_a.at[pl.ds(0, sz)],
                            acc_sh.at[pl.ds(r0 + off, sz)])
        plsc.subcore_barrier()

        def outer(b, carry):
            pltpu.sync_copy(idx_hbm.at[wid, b], idx_v)
            g = [None] * NBLK
            for j in range(2):
                g[j] = pltpu.async_copy(
                    x_hbm.at[idx_v.at[j]], rows[j % 3], gsems[j % 3])
            for j in range(NBLK):
                if j + 2 < NBLK:
                    g[j + 2] = pltpu.async_copy(
                        x_hbm.at[idx_v.at[j + 2]], rows[(j + 2) % 3],
                        gsems[(j + 2) % 3])
                g[j].wait()
                pltpu.sync_copy(rows[j % 3], acc_sh.at[idx_v.at[NBLK + j]],
                                add=True)
            return carry

        lax.fori_loop(0, nblocks, outer, 0)
        plsc.subcore_barrier()
        prev = None
        for k, (off, sz) in enumerate(slices):
            pltpu.async_copy(acc_sh.at[pl.ds(r0 + off, sz)],
                             rows[k % 2].at[pl.ds(0, sz)], gsems[k % 2]).wait()
            if prev is not None:
                prev.wait()
            prev = pltpu.async_copy(rows[k % 2].at[pl.ds(0, sz)],
                                    out_hbm.at[c, pl.ds(r0 + off, sz)],
                                    ssems[k % 2])
        prev.wait()

    return agg_kernel(x, idx4d, zeros_blk)


def _tc_layer(x, parts, W1, b1, W2, b2, gamma, beta):
    def body(x_ref, p_ref, w1_ref, b1_ref, w2_ref, b2_ref, g_ref, bt_ref, o_ref):
        h = x_ref[...] + p_ref[0, :N, :]
        for c in range(1, NCORES):
            h = h + p_ref[c, :N, :]
        t = jnp.maximum(
            jnp.dot(h, w1_ref[...], preferred_element_type=jnp.float32)
            + b1_ref[...], 0.0)
        h2 = (jnp.dot(t, w2_ref[...], preferred_element_type=jnp.float32)
              + b2_ref[...])
        mean = jnp.mean(h2, axis=0, keepdims=True)
        cen = h2 - mean
        var = jnp.mean(cen * cen, axis=0, keepdims=True)
        h3 = cen * lax.rsqrt(var + 1e-5) * g_ref[...] + bt_ref[...]
        o_ref[...] = jnp.maximum(h3, 0.0)

    return pl.pallas_call(
        body,
        out_shape=jax.ShapeDtypeStruct((N, D), jnp.float32),
    )(x, parts, W1, b1.reshape(1, D), W2, b2.reshape(1, D),
      gamma.reshape(1, D), beta.reshape(1, D))


def _tc_final(x, parts, W1, b1, W2, b2, gamma, beta, batch2d,
              Wr1, br1, Wr2, br2, Wr3p, br3p):
    def body(x_ref, p_ref, w1_ref, b1_ref, w2_ref, b2_ref, g_ref, bt_ref,
             batch_ref, wr1_ref, br1_ref, wr2_ref, br2_ref, wr3_ref, br3_ref,
             o_ref):
        h = x_ref[...] + p_ref[0, :N, :]
        for c in range(1, NCORES):
            h = h + p_ref[c, :N, :]
        t = jnp.maximum(
            jnp.dot(h, w1_ref[...], preferred_element_type=jnp.float32)
            + b1_ref[...], 0.0)
        h2 = (jnp.dot(t, w2_ref[...], preferred_element_type=jnp.float32)
              + b2_ref[...])
        mean = jnp.mean(h2, axis=0, keepdims=True)
        cen = h2 - mean
        var = jnp.mean(cen * cen, axis=0, keepdims=True)
        h3 = cen * lax.rsqrt(var + 1e-5) * g_ref[...] + bt_ref[...]
        x3 = jnp.maximum(h3, 0.0)
        seg = lax.broadcasted_iota(jnp.int32, (G, N), 0)
        onehot = (seg == batch_ref[...]).astype(jnp.float32)
        pooled = jnp.dot(onehot, x3, preferred_element_type=jnp.float32)
        r = jnp.maximum(
            jnp.dot(pooled, wr1_ref[...], preferred_element_type=jnp.float32)
            + br1_ref[...], 0.0)
        r = jnp.maximum(
            jnp.dot(r, wr2_ref[...], preferred_element_type=jnp.float32)
            + br2_ref[...], 0.0)
        o_ref[...] = (jnp.dot(r, wr3_ref[...], preferred_element_type=jnp.float32)
                      + br3_ref[...])

    return pl.pallas_call(
        body,
        out_shape=jax.ShapeDtypeStruct((G, 128), jnp.float32),
    )(x, parts, W1, b1.reshape(1, D), W2, b2.reshape(1, D),
      gamma.reshape(1, D), beta.reshape(1, D), batch2d,
      Wr1, br1.reshape(1, 512), Wr2, br2.reshape(1, 256), Wr3p, br3p)


def kernel(x, edge_index, batch,
           W1_0, b1_0, W2_0, b2_0, gamma_0, beta_0,
           W1_1, b1_1, W2_1, b2_1, gamma_1, beta_1,
           W1_2, b1_2, W2_2, b2_2, gamma_2, beta_2,
           Wr1, br1, Wr2, br2, Wr3, br3):
    E = edge_index.shape[1]
    nblocks = 10
    e_pad = NTILES * nblocks * NBLK * CHUNK
    src_p = jnp.concatenate(
        [edge_index[0], jnp.zeros((e_pad - E,), jnp.int32)])
    dst_p = jnp.concatenate(
        [edge_index[1], jnp.full((e_pad - E,), N + 200, jnp.int32)])
    src4d = src_p.reshape(NTILES, nblocks, NBLK, CHUNK)
    dst4d = dst_p.reshape(NTILES, nblocks, NBLK, CHUNK)
    idx4d = jnp.concatenate([src4d, dst4d], axis=2)
    zeros_blk = jnp.zeros((CHUNK, D), jnp.float32)
    batch2d = batch.reshape(1, N)
    T = Wr3.shape[1]
    Wr3p = jnp.pad(Wr3, ((0, 0), (0, 128 - T)))
    br3p = jnp.pad(br3, (0, 128 - T)).reshape(1, 128)

    p0 = _sc_agg(x, idx4d, zeros_blk)
    x1 = _tc_layer(x, p0, W1_0, b1_0, W2_0, b2_0, gamma_0, beta_0)
    p1 = _sc_agg(x1, idx4d, zeros_blk)
    x2 = _tc_layer(x1, p1, W1_1, b1_1, W2_1, b2_1, gamma_1, beta_1)
    p2 = _sc_agg(x2, idx4d, zeros_blk)
    out = _tc_final(x2, p2, W1_2, b1_2, W2_2, b2_2, gamma_2, beta_2,
                    batch2d, Wr1, br1, Wr2, br2, Wr3p, br3p)
    return out[:, :T]
